# Initial kernel scaffold; baseline (speedup 1.0000x reference)
import functools

import jax
import jax.numpy as jnp
from jax import lax
from jax.experimental import pallas as pl
from jax.experimental.pallas import tpu as pltpu
from jax.experimental.pallas import tpu_sc as plsc

NC, NS, L = 2, 16, 16
NW = NC * NS
R, D = 4096, 8192
ROWS_PER_W = R // NW
VECS = D // L


def _gather_body(attr_hbm, idx_hbm, out_hbm, arow, irow, orow):
    wid = lax.axis_index("s") * NC + lax.axis_index("c")
    base = wid * ROWS_PER_W

    def row_step(r, carry):
        row = base + r
        pltpu.sync_copy(attr_hbm.at[row], arow)
        pltpu.sync_copy(idx_hbm.at[row], irow)

        def vec_step(i, c2):
            idx = irow[pl.ds(i * L, L)]
            vals = plsc.load_gather(arow, [idx])
            orow[pl.ds(i * L, L)] = vals
            return c2

        lax.fori_loop(0, VECS, vec_step, 0)
        pltpu.sync_copy(orow, out_hbm.at[row])
        return carry

    lax.fori_loop(0, ROWS_PER_W, row_step, 0)


@jax.jit
def kernel(attr, batch_idxs):
    mesh = plsc.VectorSubcoreMesh(core_axis_name="c", subcore_axis_name="s")
    k = pl.kernel(
        _gather_body,
        out_type=jax.ShapeDtypeStruct((R, D), jnp.float32),
        mesh=mesh,
        scratch_types=[
            pltpu.VMEM((D,), jnp.float32),
            pltpu.VMEM((D,), jnp.int32),
            pltpu.VMEM((D,), jnp.float32),
        ],
    )
    return k(attr, batch_idxs)


# speedup vs baseline: 10.1207x; 10.1207x over previous
import jax
import jax.numpy as jnp
from jax import lax
from jax.experimental import pallas as pl
from jax.experimental.pallas import tpu as pltpu
from jax.experimental.pallas import tpu_sc as plsc

NC, NS, L = 2, 16, 16
NW = NC * NS
R, D = 4096, 8192
ROWS_PER_W = R // NW
VECS = D // L
NBUF = 2


def _gather_body(attr_hbm, idx_hbm, out_hbm,
                 arow0, arow1, irow0, irow1, orow0, orow1,
                 sin0, sin1, sout0, sout1):
    wid = lax.axis_index("s") * NC + lax.axis_index("c")
    base = wid * ROWS_PER_W
    arow = (arow0, arow1)
    irow = (irow0, irow1)
    orow = (orow0, orow1)
    sem_in = (sin0, sin1)
    sem_out = (sout0, sout1)

    pltpu.async_copy(attr_hbm.at[base], arow[0], sem_in[0])
    pltpu.async_copy(idx_hbm.at[base], irow[0], sem_in[0])

    def chunk(r0, carry):
        for b in range(NBUF):
            r = r0 * NBUF + b
            row = base + r

            @pl.when(r + 1 < ROWS_PER_W)
            def _():
                pltpu.async_copy(attr_hbm.at[row + 1], arow[b ^ 1],
                                 sem_in[b ^ 1])
                pltpu.async_copy(idx_hbm.at[row + 1], irow[b ^ 1],
                                 sem_in[b ^ 1])

            pltpu.make_async_copy(attr_hbm.at[row], arow[b],
                                  sem_in[b]).wait()
            pltpu.make_async_copy(idx_hbm.at[row], irow[b],
                                  sem_in[b]).wait()

            @pl.when(r >= 2)
            def _():
                pltpu.make_async_copy(orow[b], out_hbm.at[row - 2],
                                      sem_out[b]).wait()

            ab, ib, ob = arow[b], irow[b], orow[b]

            @plsc.parallel_loop(0, VECS, unroll=8)
            def _(i):
                idx = ib[pl.ds(i * L, L)]
                ob[pl.ds(i * L, L)] = plsc.load_gather(ab, [idx])

            pltpu.async_copy(ob, out_hbm.at[row], sem_out[b])
        return carry

    lax.fori_loop(0, ROWS_PER_W // NBUF, chunk, 0)

    pltpu.make_async_copy(orow[0], out_hbm.at[base + ROWS_PER_W - 2],
                          sem_out[0]).wait()
    pltpu.make_async_copy(orow[1], out_hbm.at[base + ROWS_PER_W - 1],
                          sem_out[1]).wait()


@jax.jit
def kernel(attr, batch_idxs):
    mesh = plsc.VectorSubcoreMesh(
        core_axis_name="c", subcore_axis_name="s", num_cores=NC, num_subcores=NS
    )
    k = pl.kernel(
        _gather_body,
        out_type=jax.ShapeDtypeStruct((R, D), jnp.float32),
        mesh=mesh,
        scratch_types=[
            pltpu.VMEM((D,), jnp.float32),
            pltpu.VMEM((D,), jnp.float32),
            pltpu.VMEM((D,), jnp.int32),
            pltpu.VMEM((D,), jnp.int32),
            pltpu.VMEM((D,), jnp.float32),
            pltpu.VMEM((D,), jnp.float32),
            pltpu.SemaphoreType.DMA,
            pltpu.SemaphoreType.DMA,
            pltpu.SemaphoreType.DMA,
            pltpu.SemaphoreType.DMA,
        ],
        compiler_params=pltpu.CompilerParams(needs_layout_passes=False),
    )
    return k(attr, batch_idxs)
